# Initial kernel scaffold; baseline (speedup 1.0000x reference)
import jax
import jax.numpy as jnp
from jax.experimental import pallas as pl


def kernel(x, adj, W1, b1, W2, b2):
    raise NotImplementedError("write your pallas kernel here")


# speedup vs baseline: 1.0126x; 1.0126x over previous
import jax
import jax.numpy as jnp
from jax.experimental import pallas as pl
from jax.experimental.pallas import tpu as pltpu

_BM = 400


def _pass1_kernel(adj_ref, x_ref, w1_ref, b1_ref, w2_ref, g_ref, xw_ref):
    i = pl.program_id(0)

    @pl.when(i == 0)
    def _():
        xb = x_ref[...].astype(jnp.bfloat16)
        w1b = w1_ref[...].astype(jnp.bfloat16)
        xw = jax.lax.dot_general(
            xb, w1b, (((1,), (1,)), ((), ())),
            preferred_element_type=jnp.float32)
        xw_ref[...] = xw.astype(jnp.bfloat16)

    a = adj_ref[...].astype(jnp.bfloat16)
    h = jnp.dot(a, xw_ref[...], preferred_element_type=jnp.float32)
    h = jnp.maximum(h + b1_ref[...], 0.0)
    w2b = w2_ref[...].astype(jnp.bfloat16)
    g = jax.lax.dot_general(
        h.astype(jnp.bfloat16), w2b, (((1,), (1,)), ((), ())),
        preferred_element_type=jnp.float32)
    g_ref[...] = g.astype(jnp.bfloat16)


def _pass2_kernel(adj_ref, g_ref, b2_ref, out_ref):
    a = adj_ref[...].astype(jnp.bfloat16)
    h = jnp.dot(a, g_ref[...], preferred_element_type=jnp.float32)
    out_ref[...] = jnp.maximum(h + b2_ref[...], 0.0)


def kernel(x, adj, W1, b1, W2, b2):
    n, d = x.shape
    h_dim = W1.shape[0]
    o_dim = W2.shape[0]
    nb = n // _BM
    b1r = b1.reshape(1, h_dim)
    b2r = b2.reshape(1, o_dim)

    g = pl.pallas_call(
        _pass1_kernel,
        grid=(nb,),
        in_specs=[
            pl.BlockSpec((_BM, n), lambda i: (i, 0)),
            pl.BlockSpec((n, d), lambda i: (0, 0)),
            pl.BlockSpec((h_dim, d), lambda i: (0, 0)),
            pl.BlockSpec((1, h_dim), lambda i: (0, 0)),
            pl.BlockSpec((o_dim, h_dim), lambda i: (0, 0)),
        ],
        out_specs=pl.BlockSpec((_BM, o_dim), lambda i: (i, 0)),
        out_shape=jax.ShapeDtypeStruct((n, o_dim), jnp.bfloat16),
        scratch_shapes=[pltpu.VMEM((n, h_dim), jnp.bfloat16)],
    )(adj, x, W1, b1r, W2)

    out = pl.pallas_call(
        _pass2_kernel,
        grid=(nb,),
        in_specs=[
            pl.BlockSpec((_BM, n), lambda i: (i, 0)),
            pl.BlockSpec((n, o_dim), lambda i: (0, 0)),
            pl.BlockSpec((1, o_dim), lambda i: (0, 0)),
        ],
        out_specs=pl.BlockSpec((_BM, o_dim), lambda i: (i, 0)),
        out_shape=jax.ShapeDtypeStruct((n, o_dim), jnp.float32),
    )(adj, g, b2r)

    return out


# speedup vs baseline: 1.0225x; 1.0097x over previous
import jax
import jax.numpy as jnp
from jax.experimental import pallas as pl
from jax.experimental.pallas import tpu as pltpu

_BM = 400


def _gcn_kernel(adj_ref, x_ref, w1_ref, b1_ref, w2_ref, b2_ref,
                out_ref, xw_ref, g_ref):
    p = pl.program_id(0)
    i = pl.program_id(1)

    @pl.when(jnp.logical_and(p == 0, i == 0))
    def _():
        xb = x_ref[...].astype(jnp.bfloat16)
        w1b = w1_ref[...].astype(jnp.bfloat16)
        xw = jax.lax.dot_general(
            xb, w1b, (((1,), (1,)), ((), ())),
            preferred_element_type=jnp.float32)
        xw_ref[...] = xw.astype(jnp.bfloat16)

    a = adj_ref[...].astype(jnp.bfloat16)

    @pl.when(p == 0)
    def _():
        h = jnp.dot(a, xw_ref[...], preferred_element_type=jnp.float32)
        h = jnp.maximum(h + b1_ref[...], 0.0)
        w2b = w2_ref[...].astype(jnp.bfloat16)
        g = jax.lax.dot_general(
            h.astype(jnp.bfloat16), w2b, (((1,), (1,)), ((), ())),
            preferred_element_type=jnp.float32)
        g_ref[pl.ds(i * _BM, _BM), :] = g.astype(jnp.bfloat16)

    @pl.when(p == 1)
    def _():
        h = jnp.dot(a, g_ref[...], preferred_element_type=jnp.float32)
        out_ref[...] = jnp.maximum(h + b2_ref[...], 0.0)


def kernel(x, adj, W1, b1, W2, b2):
    n, d = x.shape
    h_dim = W1.shape[0]
    o_dim = W2.shape[0]
    nb = n // _BM

    out = pl.pallas_call(
        _gcn_kernel,
        grid=(2, nb),
        in_specs=[
            pl.BlockSpec((_BM, n), lambda p, i: (i, 0)),
            pl.BlockSpec((n, d), lambda p, i: (0, 0)),
            pl.BlockSpec((h_dim, d), lambda p, i: (0, 0)),
            pl.BlockSpec((1, h_dim), lambda p, i: (0, 0)),
            pl.BlockSpec((o_dim, h_dim), lambda p, i: (0, 0)),
            pl.BlockSpec((1, o_dim), lambda p, i: (0, 0)),
        ],
        out_specs=pl.BlockSpec((_BM, o_dim), lambda p, i: (i * p, 0)),
        out_shape=jax.ShapeDtypeStruct((n, o_dim), jnp.float32),
        scratch_shapes=[
            pltpu.VMEM((n, h_dim), jnp.bfloat16),
            pltpu.VMEM((n, o_dim), jnp.bfloat16),
        ],
    )(adj, x, W1, b1.reshape(1, h_dim), W2, b2.reshape(1, o_dim))

    return out


# speedup vs baseline: 1.0236x; 1.0011x over previous
import jax
import jax.numpy as jnp
from jax.experimental import pallas as pl
from jax.experimental.pallas import tpu as pltpu

_BM = 400


def _gcn_kernel(adj_ref, x_ref, w1_ref, b1_ref, w2_ref, b2_ref,
                out_ref, xw_ref, g_ref):
    p = pl.program_id(0)
    i = pl.program_id(1)

    @pl.when(jnp.logical_and(p == 0, i == 0))
    def _():
        xb = x_ref[...].astype(jnp.bfloat16)
        w1b = w1_ref[...].astype(jnp.bfloat16)
        xw = jax.lax.dot_general(
            xb, w1b, (((1,), (1,)), ((), ())),
            preferred_element_type=jnp.float32)
        xw_ref[...] = xw.astype(jnp.bfloat16)

    a = adj_ref[0].astype(jnp.bfloat16)

    @pl.when(p == 0)
    def _():
        h = jnp.dot(a, xw_ref[...], preferred_element_type=jnp.float32)
        h = jnp.maximum(h + b1_ref[...], 0.0)
        w2b = w2_ref[...].astype(jnp.bfloat16)
        g = jax.lax.dot_general(
            h.astype(jnp.bfloat16), w2b, (((1,), (1,)), ((), ())),
            preferred_element_type=jnp.float32)
        g_ref[pl.ds(i * _BM, _BM), :] = g.astype(jnp.bfloat16)

    @pl.when(p == 1)
    def _():
        h = jnp.dot(a, g_ref[...], preferred_element_type=jnp.float32)
        out_ref[0] = jnp.maximum(h + b2_ref[...], 0.0)


def kernel(x, adj, W1, b1, W2, b2):
    n, d = x.shape
    h_dim = W1.shape[0]
    o_dim = W2.shape[0]
    nb = n // _BM
    adj3 = adj.reshape(nb, _BM, n)

    out = pl.pallas_call(
        _gcn_kernel,
        grid=(2, nb),
        in_specs=[
            pl.BlockSpec((1, _BM, n), lambda p, i: (i, 0, 0)),
            pl.BlockSpec((n, d), lambda p, i: (0, 0)),
            pl.BlockSpec((h_dim, d), lambda p, i: (0, 0)),
            pl.BlockSpec((1, h_dim), lambda p, i: (0, 0)),
            pl.BlockSpec((o_dim, h_dim), lambda p, i: (0, 0)),
            pl.BlockSpec((1, o_dim), lambda p, i: (0, 0)),
        ],
        out_specs=pl.BlockSpec((1, _BM, o_dim), lambda p, i: (i * p, 0, 0)),
        out_shape=jax.ShapeDtypeStruct((nb, _BM, o_dim), jnp.float32),
        scratch_shapes=[
            pltpu.VMEM((n, h_dim), jnp.bfloat16),
            pltpu.VMEM((n, o_dim), jnp.bfloat16),
        ],
    )(adj3, x, W1, b1.reshape(1, h_dim), W2, b2.reshape(1, o_dim))

    return out.reshape(n, o_dim)
